# Initial kernel scaffold; baseline (speedup 1.0000x reference)
import jax
import jax.numpy as jnp
from jax.experimental import pallas as pl


def kernel(x, W):
    raise NotImplementedError("write your pallas kernel here")


# speedup vs baseline: 1.4249x; 1.4249x over previous
import jax
import jax.numpy as jnp
from jax.experimental import pallas as pl
from jax.experimental.pallas import tpu as pltpu

_NUM_EXPERTS = 64
_TB = 1024


def _router_body(x_ref, wt_ref, i1_ref, i2_ref, w0_ref, w1_ref):
    logits = jnp.dot(x_ref[...], wt_ref[...], preferred_element_type=jnp.float32)
    iota = jax.lax.broadcasted_iota(jnp.int32, logits.shape, 1)
    m1 = jnp.max(logits, axis=1, keepdims=True)
    i1 = jnp.min(jnp.where(logits == m1, iota, _NUM_EXPERTS), axis=1, keepdims=True)
    masked = jnp.where(iota == i1, -jnp.inf, logits)
    m2 = jnp.max(masked, axis=1, keepdims=True)
    i2 = jnp.min(jnp.where(masked == m2, iota, _NUM_EXPERTS), axis=1, keepdims=True)
    e = jnp.exp(m2 - m1)
    w0 = 1.0 / (1.0 + e)
    i1_ref[...] = i1
    i2_ref[...] = i2
    w0_ref[...] = w0
    w1_ref[...] = 1.0 - w0


def kernel(x, W):
    n, d = x.shape
    grid = (n // _TB,)
    out1 = jax.ShapeDtypeStruct((n, 1), jnp.int32)
    outf = jax.ShapeDtypeStruct((n, 1), jnp.float32)
    i1, i2, w0, w1 = pl.pallas_call(
        _router_body,
        grid=grid,
        in_specs=[
            pl.BlockSpec((_TB, d), lambda i: (i, 0)),
            pl.BlockSpec((d, _NUM_EXPERTS), lambda i: (0, 0)),
        ],
        out_specs=[
            pl.BlockSpec((_TB, 1), lambda i: (i, 0)),
            pl.BlockSpec((_TB, 1), lambda i: (i, 0)),
            pl.BlockSpec((_TB, 1), lambda i: (i, 0)),
            pl.BlockSpec((_TB, 1), lambda i: (i, 0)),
        ],
        out_shape=[out1, out1, outf, outf],
        compiler_params=pltpu.CompilerParams(
            dimension_semantics=("arbitrary",),
        ),
    )(x, W.T)
    topi = jnp.concatenate([i1, i2], axis=1)
    weights = jnp.concatenate([w0, w1], axis=1)
    return (topi, weights)


# speedup vs baseline: 1.5697x; 1.1016x over previous
import functools

import jax
import jax.numpy as jnp
from jax import lax
from jax.experimental import pallas as pl
from jax.experimental.pallas import tpu as pltpu
from jax.experimental.pallas import tpu_sc as plsc

_E = 64
_TB = 1024
_NC = 2
_NS = 16
_NW = _NC * _NS
_L = 16


def _matmul_body(x_ref, wt_ref, lg_ref):
    lg_ref[...] = jnp.dot(x_ref[...], wt_ref[...],
                          preferred_element_type=jnp.float32).T


def _gate_logits_t(x, WT):
    n, d = x.shape
    return pl.pallas_call(
        _matmul_body,
        grid=(n // _TB,),
        in_specs=[
            pl.BlockSpec((_TB, d), lambda i: (i, 0)),
            pl.BlockSpec((d, _E), lambda i: (0, 0)),
        ],
        out_specs=pl.BlockSpec((_E, _TB), lambda i: (0, i)),
        out_shape=jax.ShapeDtypeStruct((_E, n), jnp.float32),
        compiler_params=pltpu.CompilerParams(
            dimension_semantics=("arbitrary",),
        ),
    )(x, WT)


def _sc_top2_body(tpw, lg_hbm, i1_hbm, i2_hbm, w0_hbm, w1_hbm,
                  lg_v, i1_v, i2_v, w0_v, w1_v):
    wid = lax.axis_index("s") * _NC + lax.axis_index("c")
    base = wid * tpw
    pltpu.sync_copy(lg_hbm.at[:, pl.ds(base, tpw)], lg_v)

    def group(g, carry):
        tok = g * _L
        neg = jnp.full((_L,), -jnp.inf, jnp.float32)
        zero = jnp.zeros((_L,), jnp.int32)
        bestv, secondv = neg, neg
        besti, secondi = zero, zero
        for e in range(_E):
            col = jnp.full((_L,), e, jnp.int32)
            v = lg_v[e, pl.ds(tok, _L)]
            gt1 = v > bestv
            gt2 = v > secondv
            sv = jnp.where(gt2, v, secondv)
            si = jnp.where(gt2, col, secondi)
            secondv = jnp.where(gt1, bestv, sv)
            secondi = jnp.where(gt1, besti, si)
            bestv = jnp.where(gt1, v, bestv)
            besti = jnp.where(gt1, col, besti)
        ex = jnp.exp(secondv - bestv)
        w0 = 1.0 / (1.0 + ex)
        i1_v[pl.ds(tok, _L)] = besti
        i2_v[pl.ds(tok, _L)] = secondi
        w0_v[pl.ds(tok, _L)] = w0
        w1_v[pl.ds(tok, _L)] = 1.0 - w0
        return carry

    lax.fori_loop(0, tpw // _L, group, 0)
    pltpu.sync_copy(i1_v, i1_hbm.at[pl.ds(base, tpw)])
    pltpu.sync_copy(i2_v, i2_hbm.at[pl.ds(base, tpw)])
    pltpu.sync_copy(w0_v, w0_hbm.at[pl.ds(base, tpw)])
    pltpu.sync_copy(w1_v, w1_hbm.at[pl.ds(base, tpw)])


def _sc_top2(logits_t):
    n = logits_t.shape[1]
    tpw = n // _NW
    mesh = plsc.VectorSubcoreMesh(core_axis_name="c", subcore_axis_name="s")
    out_i = jax.ShapeDtypeStruct((n,), jnp.int32)
    out_f = jax.ShapeDtypeStruct((n,), jnp.float32)
    fn = pl.kernel(
        functools.partial(_sc_top2_body, tpw),
        out_type=[out_i, out_i, out_f, out_f],
        mesh=mesh,
        scratch_types=[
            pltpu.VMEM((_E, tpw), jnp.float32),
            pltpu.VMEM((tpw,), jnp.int32),
            pltpu.VMEM((tpw,), jnp.int32),
            pltpu.VMEM((tpw,), jnp.float32),
            pltpu.VMEM((tpw,), jnp.float32),
        ],
    )
    return fn(logits_t)


def kernel(x, W):
    logits_t = _gate_logits_t(x, W.T)
    i1, i2, w0, w1 = _sc_top2(logits_t)
    topi = jnp.stack([i1, i2], axis=1)
    weights = jnp.stack([w0, w1], axis=1)
    return (topi, weights)
